# baseline (speedup 1.0000x reference)
import jax
import jax.numpy as jnp
from jax import lax
from jax.experimental import pallas as pl
from jax.experimental.pallas import tpu as pltpu
from jax.experimental.pallas import tpu_sc as plsc

D = 64
S = 32
NQ = S * S
ROWS = NQ + 1
MAXR = 14
NBLK = 64
NC = 2
NS = 16
L = 16
NBUF = 4
BPT = NBLK // NS


def _sc_body(v_hbm, h_hbm, out_hbm, vtab, htab, hpat, bbuf, cbuf,
             shblk, shc, sem):
    c = lax.axis_index("c")
    sid = lax.axis_index("s")
    w = sid * NC + c

    pltpu.sync_copy(v_hbm, vtab)
    pltpu.sync_copy(h_hbm, htab)

    for r in range(D // L):
        cval = vtab[0, pl.ds(r * L, L)] + htab[0, pl.ds(r * L, L)]
        for t in range(S + 1):
            cbuf[t, pl.ds(r * L, L)] = cval

    @pl.when(sid == 0)
    def _():
        pltpu.sync_copy(cbuf, shc)

    plsc.subcore_barrier()

    row0 = pltpu.async_copy(shc, out_hbm.at[0, pl.ds(w * S, S + 1)], sem)

    inflight = [None] * NBUF
    for step in range(NS):
        m = 2 * step + c
        buf = step % NBUF

        if inflight[buf] is not None:
            for h in inflight[buf]:
                h.wait()
        plsc.subcore_barrier()

        def _hrow(t, carry):
            hidx = jnp.clip(t - m, -MAXR, MAXR) + MAXR + 1
            for r in range(D // L):
                hpat[t, pl.ds(r * L, L)] = htab[hidx, pl.ds(r * L, L)]
            return carry

        lax.fori_loop(0, S, _hrow, 0)

        for b in range(BPT):
            jb = sid * BPT + b
            vidx = jnp.clip(jb - (S - 1), -MAXR, MAXR) + MAXR + 1

            def _brow(t, carry):
                for r in range(D // L):
                    bbuf[b * S + t, pl.ds(r * L, L)] = (
                        vtab[vidx, pl.ds(r * L, L)] + hpat[t, pl.ds(r * L, L)])
                return carry

            lax.fori_loop(0, S, _brow, 0)

        pltpu.sync_copy(bbuf, shblk.at[buf, pl.ds(sid * BPT * S, BPT * S)])
        plsc.subcore_barrier()

        handles = []
        for r in range(2):
            a = 2 * sid + r
            i = a * S + m + 1
            handles.append(pltpu.async_copy(
                shblk.at[buf, pl.ds((S - 1 - a) * S, NQ)],
                out_hbm.at[i, pl.ds(1, NQ)], sem))
            handles.append(pltpu.async_copy(
                shc.at[pl.ds(0, 1)], out_hbm.at[i, pl.ds(0, 1)], sem))
        inflight[buf] = handles

    for handles in inflight:
        if handles is not None:
            for h in handles:
                h.wait()
    row0.wait()


@jax.jit
def _expand(v, h):
    mesh = plsc.VectorSubcoreMesh(core_axis_name="c", subcore_axis_name="s")
    return pl.kernel(
        _sc_body,
        out_type=jax.ShapeDtypeStruct((ROWS, ROWS, D), jnp.float32),
        mesh=mesh,
        compiler_params=pltpu.CompilerParams(use_tc_tiling_on_sc=False),
        scratch_types=[
            pltpu.VMEM((2 * MAXR + 2, D), jnp.float32),
            pltpu.VMEM((2 * MAXR + 2, D), jnp.float32),
            pltpu.VMEM((S, D), jnp.float32),
            pltpu.VMEM((BPT * S, D), jnp.float32),
            pltpu.VMEM((S + 1, D), jnp.float32),
            pltpu.VMEM_SHARED((NBUF, NBLK * S, D), jnp.float32),
            pltpu.VMEM_SHARED((S + 1, D), jnp.float32),
            pltpu.SemaphoreType.DMA,
        ],
    )(v, h)


def kernel(embeddings_table_v, embeddings_table_h, length_q, length_k):
    del length_q, length_k
    return _expand(embeddings_table_v, embeddings_table_h)


# speedup vs baseline: 1.6000x; 1.6000x over previous
import jax
import jax.numpy as jnp
from jax import lax
from jax.experimental import pallas as pl
from jax.experimental.pallas import tpu as pltpu
from jax.experimental.pallas import tpu_sc as plsc

D = 64
S = 32
NQ = S * S
ROWS = NQ + 1
MAXR = 14
NC = 2
NS = 16
L = 16


def _sc_gather(v_hbm, h_hbm, vexp_hbm, hpats_hbm, vtab, htab, hbuf, vbufa,
               vbufb, sem):
    w = lax.axis_index("s") * NC + lax.axis_index("c")

    pltpu.sync_copy(v_hbm, vtab)
    pltpu.sync_copy(h_hbm, htab)

    def _rows(t, carry):
        hidx = jnp.clip(t - w, -MAXR, MAXR) + MAXR + 1
        vidxa = jnp.clip(t - w, -MAXR, MAXR) + MAXR + 1
        vidxb = jnp.clip(t - w - 1, -MAXR, MAXR) + MAXR + 1
        for r in range(D // L):
            sl = pl.ds(r * L, L)
            hbuf[t, sl] = htab[hidx, sl]
            vbufa[t, sl] = vtab[vidxa, sl]
            vbufb[t, sl] = vtab[vidxb, sl]
        return carry

    lax.fori_loop(0, S, _rows, 0)

    copies = [
        pltpu.async_copy(hbuf, hpats_hbm.at[w], sem),
        pltpu.async_copy(vbufa, vexp_hbm.at[w], sem),
        pltpu.async_copy(vbufb, vexp_hbm.at[w + 1], sem),
    ]
    for c in copies:
        c.wait()


def _tc_fanout(vexp_ref, hpats_ref, v_ref, h_ref, out_ref):
    b = pl.program_id(0)
    const = v_ref[0:1, :] + h_ref[0:1, :]

    vwin = vexp_ref[b]
    vwin0 = vexp_ref[jnp.maximum(b, 1) - 1]

    hp31 = hpats_ref[S - 1]
    row0 = (vwin0[:, None, :] + hp31[None, :, :]).reshape(NQ, D)
    row0 = jnp.where(b == 0, jnp.broadcast_to(const, (NQ, D)), row0)
    out_ref[0, :, :] = jnp.concatenate([const, row0], axis=0)

    for r in range(1, S):
        row = (vwin[:, None, :] + hpats_ref[r - 1][None, :, :]).reshape(NQ, D)
        out_ref[r, :, :] = jnp.concatenate([const, row], axis=0)


@jax.jit
def _expand(v, h):
    mesh = plsc.VectorSubcoreMesh(core_axis_name="c", subcore_axis_name="s")
    vexp, hpats = pl.kernel(
        _sc_gather,
        out_type=(
            jax.ShapeDtypeStruct((S + 1, S, D), jnp.float32),
            jax.ShapeDtypeStruct((S, S, D), jnp.float32),
        ),
        mesh=mesh,
        compiler_params=pltpu.CompilerParams(use_tc_tiling_on_sc=False),
        scratch_types=[
            pltpu.VMEM((2 * MAXR + 2, D), jnp.float32),
            pltpu.VMEM((2 * MAXR + 2, D), jnp.float32),
            pltpu.VMEM((S, D), jnp.float32),
            pltpu.VMEM((S, D), jnp.float32),
            pltpu.VMEM((S, D), jnp.float32),
            pltpu.SemaphoreType.DMA,
        ],
    )(v, h)

    vp = jnp.zeros((S, D), jnp.float32).at[: 2 * MAXR + 2].set(v)
    hp = jnp.zeros((S, D), jnp.float32).at[: 2 * MAXR + 2].set(h)

    return pl.pallas_call(
        _tc_fanout,
        out_shape=jax.ShapeDtypeStruct((ROWS, ROWS, D), jnp.float32),
        grid=(S + 1,),
        in_specs=[
            pl.BlockSpec((S + 1, S, D), lambda b: (0, 0, 0)),
            pl.BlockSpec((S, S, D), lambda b: (0, 0, 0)),
            pl.BlockSpec((S, D), lambda b: (0, 0)),
            pl.BlockSpec((S, D), lambda b: (0, 0)),
        ],
        out_specs=pl.BlockSpec((S, ROWS, D), lambda b: (b, 0, 0)),
    )(vexp, hpats, vp, hp)


def kernel(embeddings_table_v, embeddings_table_h, length_q, length_k):
    del length_q, length_k
    return _expand(embeddings_table_v, embeddings_table_h)


# speedup vs baseline: 1.6026x; 1.0016x over previous
import jax
import jax.numpy as jnp
from jax import lax
from jax.experimental import pallas as pl
from jax.experimental.pallas import tpu as pltpu
from jax.experimental.pallas import tpu_sc as plsc

D = 64
S = 32
NQ = S * S
ROWS = NQ + 1
MAXR = 14
NC = 2
NS = 16
L = 16


def _sc_gather(v_hbm, h_hbm, vexp_hbm, hpats_hbm, vtab, htab, hbuf, vbufa,
               vbufb, sem):
    w = lax.axis_index("s") * NC + lax.axis_index("c")

    pltpu.sync_copy(v_hbm, vtab)
    pltpu.sync_copy(h_hbm, htab)

    def _rows(t, carry):
        hidx = jnp.clip(t - w, -MAXR, MAXR) + MAXR + 1
        vidxa = jnp.clip(t - w, -MAXR, MAXR) + MAXR + 1
        vidxb = jnp.clip(t - w - 1, -MAXR, MAXR) + MAXR + 1
        for r in range(D // L):
            sl = pl.ds(r * L, L)
            hbuf[t, sl] = htab[hidx, sl]
            vbufa[t, sl] = vtab[vidxa, sl]
            vbufb[t, sl] = vtab[vidxb, sl]
        return carry

    lax.fori_loop(0, S, _rows, 0)

    copies = [
        pltpu.async_copy(hbuf, hpats_hbm.at[w], sem),
        pltpu.async_copy(vbufa, vexp_hbm.at[w], sem),
        pltpu.async_copy(vbufb, vexp_hbm.at[w + 1], sem),
    ]
    for c in copies:
        c.wait()


def _tc_fanout(vexp_ref, hpats_ref, v_ref, h_ref, out_ref):
    b = pl.program_id(0)
    const = v_ref[0:1, :] + h_ref[0:1, :]

    vwin = vexp_ref[b]
    vwin0 = vexp_ref[jnp.maximum(b, 1) - 1]

    hp31 = hpats_ref[S - 1]
    row0 = (vwin0[:, None, :] + hp31[None, :, :]).reshape(NQ, D)
    row0 = jnp.where(b == 0, jnp.broadcast_to(const, (NQ, D)), row0)
    out_ref[0, :, :] = jnp.concatenate([const, row0], axis=0)

    for r in range(1, S):
        row = (vwin[:, None, :] + hpats_ref[r - 1][None, :, :]).reshape(NQ, D)
        out_ref[r, :, :] = jnp.concatenate([const, row], axis=0)


@jax.jit
def _expand(v, h):
    vp = jnp.zeros((S, D), jnp.float32).at[: 2 * MAXR + 2].set(v)
    hp = jnp.zeros((S, D), jnp.float32).at[: 2 * MAXR + 2].set(h)

    mesh = plsc.VectorSubcoreMesh(core_axis_name="c", subcore_axis_name="s")
    vexp, hpats = pl.kernel(
        _sc_gather,
        out_type=(
            jax.ShapeDtypeStruct((S + 1, S, D), jnp.float32),
            jax.ShapeDtypeStruct((S, S, D), jnp.float32),
        ),
        mesh=mesh,
        compiler_params=pltpu.CompilerParams(use_tc_tiling_on_sc=True),
        scratch_types=[
            pltpu.VMEM((S, D), jnp.float32),
            pltpu.VMEM((S, D), jnp.float32),
            pltpu.VMEM((S, D), jnp.float32),
            pltpu.VMEM((S, D), jnp.float32),
            pltpu.VMEM((S, D), jnp.float32),
            pltpu.SemaphoreType.DMA,
        ],
    )(vp, hp)

    return pl.pallas_call(
        _tc_fanout,
        out_shape=jax.ShapeDtypeStruct((ROWS, ROWS, D), jnp.float32),
        grid=(S + 1,),
        in_specs=[
            pl.BlockSpec((S + 1, S, D), lambda b: (0, 0, 0)),
            pl.BlockSpec((S, S, D), lambda b: (0, 0, 0)),
            pl.BlockSpec((S, D), lambda b: (0, 0)),
            pl.BlockSpec((S, D), lambda b: (0, 0)),
        ],
        out_specs=pl.BlockSpec((S, ROWS, D), lambda b: (b, 0, 0)),
    )(vexp, hpats, vp, hp)


def kernel(embeddings_table_v, embeddings_table_h, length_q, length_k):
    del length_q, length_k
    return _expand(embeddings_table_v, embeddings_table_h)


# speedup vs baseline: 5.8368x; 3.6421x over previous
import jax
import jax.numpy as jnp
from jax import lax
from jax.experimental import pallas as pl
from jax.experimental.pallas import tpu as pltpu
from jax.experimental.pallas import tpu_sc as plsc

D = 64
S = 32
NQ = S * S
ROWS = NQ + 1
MAXR = 14
NC = 2
NS = 16
L = 16
DCH = 16


def _sc_gather(v_hbm, h_hbm, vrepf_hbm, hrepf_hbm, vtab, htab, vb0, vb1,
               hb0, hb1, sem):
    w = lax.axis_index("s") * NC + lax.axis_index("c")

    pltpu.sync_copy(v_hbm, vtab)
    pltpu.sync_copy(h_hbm, htab)

    t_lo = lax.iota(jnp.int32, L)
    hrow_lo = jnp.clip(t_lo - w, -MAXR, MAXR) + MAXR + 1
    hrow_hi = jnp.clip(t_lo + L - w, -MAXR, MAXR) + MAXR + 1
    zero16 = jnp.zeros((L,), jnp.int32)
    lane0 = t_lo == 0

    vbufs = (vb0, vb1)
    hbufs = (hb0, hb1)
    handles = [None, None]
    for chunk in range(D // DCH):
        vb = vbufs[chunk % 2]
        hb = hbufs[chunk % 2]
        if handles[chunk % 2] is not None:
            for h in handles[chunk % 2]:
                h.wait()

        def _drow(dloc, carry, chunk=chunk, vb=vb, hb=hb):
            d = chunk * DCH + dloc
            dcol = jnp.full((L,), d, jnp.int32)
            hv_lo = plsc.load_gather(htab, [hrow_lo, dcol])
            hv_hi = plsc.load_gather(htab, [hrow_hi, dcol])
            cv = (plsc.load_gather(vtab, [zero16, dcol])
                  + plsc.load_gather(htab, [zero16, dcol]))
            drow = jnp.full((L,), dloc, jnp.int32)
            plsc.store_scatter(hb, [drow, zero16], cv, mask=lane0)
            plsc.store_scatter(vb, [drow, zero16],
                               jnp.zeros((L,), jnp.float32), mask=lane0)
            for b in range(S):
                vidx = jnp.clip(b - w, -MAXR, MAXR) + MAXR + 1
                vrow = jnp.full((L,), vidx, jnp.int32)
                vv = plsc.load_gather(vtab, [vrow, dcol])
                vb[dloc, pl.ds(1 + S * b, L)] = vv
                vb[dloc, pl.ds(1 + S * b + L, L)] = vv
                hb[dloc, pl.ds(1 + S * b, L)] = hv_lo
                hb[dloc, pl.ds(1 + S * b + L, L)] = hv_hi
            return carry

        lax.fori_loop(0, DCH, _drow, 0)
        handles[chunk % 2] = [
            pltpu.async_copy(
                vb, vrepf_hbm.at[w, pl.ds(chunk * DCH, DCH)], sem),
            pltpu.async_copy(
                hb, hrepf_hbm.at[w, pl.ds(chunk * DCH, DCH)], sem),
        ]

    for hs in handles:
        if hs is not None:
            for h in hs:
                h.wait()


def _tc_fanout(vrepf_ref, hrepf_ref, out_ref):
    g = pl.program_id(0)
    vwin = vrepf_ref[jnp.minimum(g, S - 1)]
    vwin0 = vrepf_ref[jnp.maximum(g, 1) - 1]
    hp31 = hrepf_ref[S - 1]

    const_row = jnp.broadcast_to(hp31[:, 0:1], (D, ROWS))
    row0 = jnp.where(g == 0, const_row, vwin0 + hp31)
    out_ref[0, :, :] = row0

    for r in range(1, S):
        out_ref[r, :, :] = vwin + hrepf_ref[r - 1]


@jax.jit
def _expand(v, h):
    vp = jnp.zeros((S, D), jnp.float32).at[: 2 * MAXR + 2].set(v)
    hp = jnp.zeros((S, D), jnp.float32).at[: 2 * MAXR + 2].set(h)

    mesh = plsc.VectorSubcoreMesh(core_axis_name="c", subcore_axis_name="s")
    vrepf, hrepf = pl.kernel(
        _sc_gather,
        out_type=(
            jax.ShapeDtypeStruct((S, D, ROWS), jnp.float32),
            jax.ShapeDtypeStruct((S, D, ROWS), jnp.float32),
        ),
        mesh=mesh,
        compiler_params=pltpu.CompilerParams(
            use_tc_tiling_on_sc=False, needs_layout_passes=False),
        scratch_types=[
            pltpu.VMEM((S, D), jnp.float32),
            pltpu.VMEM((S, D), jnp.float32),
            pltpu.VMEM((DCH, ROWS), jnp.float32),
            pltpu.VMEM((DCH, ROWS), jnp.float32),
            pltpu.VMEM((DCH, ROWS), jnp.float32),
            pltpu.VMEM((DCH, ROWS), jnp.float32),
            pltpu.SemaphoreType.DMA,
        ],
    )(vp, hp)

    out_t = pl.pallas_call(
        _tc_fanout,
        out_shape=jax.ShapeDtypeStruct((ROWS, D, ROWS), jnp.float32),
        grid=(S + 1,),
        in_specs=[
            pl.BlockSpec((S, D, ROWS), lambda g: (0, 0, 0)),
            pl.BlockSpec((S, D, ROWS), lambda g: (0, 0, 0)),
        ],
        out_specs=pl.BlockSpec((S, D, ROWS), lambda g: (g, 0, 0)),
    )(vrepf, hrepf)

    return jnp.transpose(out_t, (0, 2, 1))


def kernel(embeddings_table_v, embeddings_table_h, length_q, length_k):
    del length_q, length_k
    return _expand(embeddings_table_v, embeddings_table_h)
